# baseline (speedup 1.0000x reference)
import functools

import jax
import jax.numpy as jnp
from jax import lax
from jax.experimental import pallas as pl
from jax.experimental.pallas import tpu as pltpu
from jax.experimental.pallas import tpu_sc as plsc

_B, _S, _H = 4, 4096, 2048
_N = _B * _S
_F = 12288
_NW = 32
_RPW = (_N - _F) // _NW
_C = 16
_G = _RPW // _C
_L = 16
_NV = _H // _L


def _tc_body(rows, alpha_ref, h_ref, m_ref, scale_ref, shift_ref, out_ref):
    i = pl.program_id(0)
    h = h_ref[...]
    m = m_ref[pl.ds(i * rows, rows), :]
    t = jnp.where(m == 2, alpha_ref[0], jnp.float32(0.0))
    out_ref[...] = h + t * (h * scale_ref[...] + shift_ref[...])


def _tc_call(h, m, scale2, shift2, alpha1):
    rows = 1024
    return pl.pallas_call(
        functools.partial(_tc_body, rows),
        grid=(_F // rows,),
        in_specs=[
            pl.BlockSpec(memory_space=pltpu.SMEM),
            pl.BlockSpec((rows, _H), lambda i: (i, 0)),
            pl.BlockSpec((_F, 1), lambda i: (0, 0)),
            pl.BlockSpec((1, _H), lambda i: (0, 0)),
            pl.BlockSpec((1, _H), lambda i: (0, 0)),
        ],
        out_specs=pl.BlockSpec((rows, _H), lambda i: (i, 0)),
        out_shape=jax.ShapeDtypeStruct((_F, _H), jnp.float32),
    )(alpha1, h, m, scale2, shift2)


def _sc_body(h_hbm, m_hbm, scale_hbm, shift_hbm, alpha_hbm, out_hbm,
             buf0, buf1, a_v, b_v, scale_v, shift_v, alpha_v, mask_v,
             ld_sem, st_sem):
    nc = 2
    wid = lax.axis_index("s") * nc + lax.axis_index("c")
    src_base = _F + wid * _RPW
    dst_base = wid * _RPW

    pltpu.sync_copy(m_hbm.at[pl.ds(src_base, _RPW)], mask_v)
    pltpu.sync_copy(scale_hbm, scale_v)
    pltpu.sync_copy(shift_hbm, shift_v)
    pltpu.sync_copy(alpha_hbm, alpha_v)
    alpha = alpha_v[...][0]

    def _fold(j, carry):
        sl = pl.ds(j * _L, _L)
        a_v[sl] = scale_v[sl] * alpha + 1.0
        b_v[sl] = shift_v[sl] * alpha
        return carry
    lax.fori_loop(0, _NV, _fold, 0)

    def _compute(g, buf):
        mv = mask_v[pl.ds(g * _C, _C)]
        for r in range(_C):
            @pl.when(mv[r] == 2)
            def _fix(r=r):
                def _vec(j, carry):
                    sl = pl.ds(j * _L, _L)
                    buf[r, sl] = buf[r, sl] * a_v[sl] + b_v[sl]
                    return carry
                lax.fori_loop(0, _NV, _vec, 0)

    def _ld(g, buf):
        return pltpu.async_copy(
            h_hbm.at[pl.ds(src_base + g * _C, _C)], buf, ld_sem)

    def _st(g, buf):
        return pltpu.async_copy(
            buf, out_hbm.at[pl.ds(dst_base + g * _C, _C)], st_sem)

    def _wait_ld():
        pltpu.make_async_copy(
            h_hbm.at[pl.ds(src_base, _C)], buf0, ld_sem).wait()

    def _wait_st():
        pltpu.make_async_copy(
            buf0, out_hbm.at[pl.ds(dst_base, _C)], st_sem).wait()

    K = _G // 2
    _ld(0, buf0)

    def _step(k, carry):
        g0 = 2 * k

        @pl.when(k >= 1)
        def _drain1():
            _wait_st()
        _ld(g0 + 1, buf1)
        _wait_ld()
        _compute(g0, buf0)
        _st(g0, buf0)

        @pl.when(k < K - 1)
        def _next0():
            _wait_st()
            _ld(g0 + 2, buf0)
        _wait_ld()
        _compute(g0 + 1, buf1)
        _st(g0 + 1, buf1)
        return carry

    lax.fori_loop(0, K, _step, 0)
    _wait_st()
    _wait_st()


def _sc_call(h, m, scale, shift, alpha16):
    mesh = plsc.VectorSubcoreMesh(core_axis_name="c", subcore_axis_name="s")
    return pl.kernel(
        _sc_body,
        out_type=jax.ShapeDtypeStruct((_N - _F, _H), jnp.float32),
        mesh=mesh,
        scratch_types=[
            pltpu.VMEM((_C, _H), jnp.float32),
            pltpu.VMEM((_C, _H), jnp.float32),
            pltpu.VMEM((_H,), jnp.float32),
            pltpu.VMEM((_H,), jnp.float32),
            pltpu.VMEM((_H,), jnp.float32),
            pltpu.VMEM((_H,), jnp.float32),
            pltpu.VMEM((_L,), jnp.float32),
            pltpu.VMEM((_RPW,), jnp.int32),
            pltpu.SemaphoreType.DMA,
            pltpu.SemaphoreType.DMA,
        ],
    )(h, m, scale, shift, alpha16)


def kernel(hidden_states, input_mask, scale, shift, alpha):
    h = hidden_states.reshape(_N, _H)
    m2 = input_mask.reshape(_N, 1)
    m1 = input_mask.reshape(_N)
    scale2 = scale.reshape(1, _H)
    shift2 = shift.reshape(1, _H)
    alpha1 = jnp.asarray(alpha, jnp.float32).reshape(1)
    alpha16 = jnp.broadcast_to(alpha1, (_L,))

    sc_out = _sc_call(h, m1, scale, shift, alpha16)
    tc_out = _tc_call(h, m2, scale2, shift2, alpha1)
    out = jnp.concatenate([tc_out, sc_out], axis=0)
    return out.reshape(_B, _S, _H)


# speedup vs baseline: 2.1280x; 2.1280x over previous
import jax
import jax.numpy as jnp
from jax import lax
from jax.experimental import pallas as pl
from jax.experimental.pallas import tpu as pltpu

_B, _S, _H = 4, 4096, 2048
_N = _B * _S
_CHR = 512
_STEPS = _N // _CHR
_NBUF = 8


def _body(alpha_ref, h_hbm, m_ref, scale_ref, shift_ref, out_hbm,
          bufs, ld_sem, st_sem):
    def _ld(c, b):
        pltpu.make_async_copy(
            h_hbm.at[pl.ds(c * _CHR, _CHR), :], bufs[b], ld_sem.at[b]).start()

    def _st(c, b):
        pltpu.make_async_copy(
            bufs[b], out_hbm.at[pl.ds(c * _CHR, _CHR), :], st_sem.at[b]).start()

    def _wait_ld(b):
        pltpu.make_async_copy(
            h_hbm.at[pl.ds(0, _CHR), :], bufs[b], ld_sem.at[b]).wait()

    def _wait_st(b):
        pltpu.make_async_copy(
            bufs[b], out_hbm.at[pl.ds(0, _CHR), :], st_sem.at[b]).wait()

    for i in range(_NBUF - 1):
        _ld(i, i)

    alpha = alpha_ref[0]
    scale_row = scale_ref[...]
    shift_row = shift_ref[...]

    def _outer(k2, carry):
        c0 = k2 * _NBUF
        for b in range(_NBUF):
            c = c0 + b
            _wait_ld(b)
            h = bufs[b][...]
            t = jnp.where(m_ref[pl.ds(c * _CHR, _CHR), :] == 2,
                          alpha, jnp.float32(0.0))
            bufs[b][...] = h + t * (h * scale_row + shift_row)
            _st(c, b)

            @pl.when(c + _NBUF - 1 < _STEPS)
            def _prefetch(c=c, b=b):
                @pl.when(c >= 1)
                def _drain():
                    _wait_st((b - 1) % _NBUF)
                _ld(c + _NBUF - 1, (b - 1) % _NBUF)
        return carry

    lax.fori_loop(0, _STEPS // _NBUF, _outer, 0, unroll=False)
    for b in range(_NBUF):
        _wait_st(b)


def kernel(hidden_states, input_mask, scale, shift, alpha):
    h = hidden_states.reshape(_N, _H)
    m = input_mask.reshape(_N, 1)
    scale2 = scale.reshape(1, _H)
    shift2 = shift.reshape(1, _H)
    alpha1 = jnp.asarray(alpha, jnp.float32).reshape(1)

    out = pl.pallas_call(
        _body,
        in_specs=[
            pl.BlockSpec(memory_space=pltpu.SMEM),
            pl.BlockSpec(memory_space=pl.ANY),
            pl.BlockSpec(memory_space=pltpu.VMEM),
            pl.BlockSpec(memory_space=pltpu.VMEM),
            pl.BlockSpec(memory_space=pltpu.VMEM),
        ],
        out_specs=pl.BlockSpec(memory_space=pl.ANY),
        out_shape=jax.ShapeDtypeStruct((_N, _H), jnp.float32),
        scratch_shapes=[
            [pltpu.VMEM((_CHR, _H), jnp.float32) for _ in range(_NBUF)],
            pltpu.SemaphoreType.DMA((_NBUF,)),
            pltpu.SemaphoreType.DMA((_NBUF,)),
        ],
    )(alpha1, h, m, scale2, shift2)
    return out.reshape(_B, _S, _H)


# speedup vs baseline: 2.1281x; 1.0000x over previous
import functools

import jax
import jax.numpy as jnp
from jax.experimental import pallas as pl
from jax.experimental.pallas import tpu as pltpu

_B, _S, _H = 4, 4096, 2048
_N = _B * _S
_ROWS = 1024


def _body(alpha_ref, h_ref, m_ref, scale_ref, shift_ref, out_ref):
    i = pl.program_id(0)
    h = h_ref[...]
    m = m_ref[pl.ds(i * _ROWS, _ROWS), :]
    t = jnp.where(m == 2, alpha_ref[0], jnp.float32(0.0))
    out_ref[...] = h + t * (h * scale_ref[...] + shift_ref[...])


def kernel(hidden_states, input_mask, scale, shift, alpha):
    h = hidden_states.reshape(_N, _H)
    m = input_mask.reshape(_N, 1)
    scale2 = scale.reshape(1, _H)
    shift2 = shift.reshape(1, _H)
    alpha1 = jnp.asarray(alpha, jnp.float32).reshape(1)

    out = pl.pallas_call(
        _body,
        grid=(_N // _ROWS,),
        in_specs=[
            pl.BlockSpec(memory_space=pltpu.SMEM),
            pl.BlockSpec((_ROWS, _H), lambda i: (i, 0)),
            pl.BlockSpec((_N, 1), lambda i: (0, 0)),
            pl.BlockSpec((1, _H), lambda i: (0, 0)),
            pl.BlockSpec((1, _H), lambda i: (0, 0)),
        ],
        out_specs=pl.BlockSpec((_ROWS, _H), lambda i: (i, 0)),
        out_shape=jax.ShapeDtypeStruct((_N, _H), jnp.float32),
    )(alpha1, h, m, scale2, shift2)
    return out.reshape(_B, _S, _H)
